# baseline (speedup 1.0000x reference)
import functools

import jax
import jax.numpy as jnp
from jax.experimental import pallas as pl
from jax.experimental.pallas import tpu as pltpu


def _tc_broadcast_rows(pe2d, batch, length):
    d = pe2d.shape[1]
    bs = 512
    grid = -(-length // bs)

    def body(pe_ref, out_ref):
        out_ref[...] = pe_ref[...][None]

    return pl.pallas_call(
        body,
        grid=(grid, batch),
        in_specs=[pl.BlockSpec((bs, d), lambda i, b: (i, 0))],
        out_specs=pl.BlockSpec((1, bs, d), lambda i, b: (b, i, 0)),
        out_shape=jax.ShapeDtypeStruct((batch, length, d), jnp.float32),
        compiler_params=pltpu.CompilerParams(
            dimension_semantics=("arbitrary", "arbitrary"),
        ),
    )(pe2d)


def kernel(x, pe):
    b, s, _ = x.shape
    length = 2 * s - 1
    return _tc_broadcast_rows(pe[0], b, length)


# speedup vs baseline: 1.1388x; 1.1388x over previous
import functools

import jax
import jax.numpy as jnp
from jax.experimental import pallas as pl
from jax.experimental.pallas import tpu as pltpu


def _tc_broadcast_rows(pe2d, batch, length):
    d = pe2d.shape[1]
    chunk = 1024
    nch = -(-length // chunk)
    nbuf = 3

    tail = length - (nch - 1) * chunk

    def body(pe_hbm, out_hbm, bufs, tbuf, gsems, wsems):
        def gcopy(i):
            return pltpu.make_async_copy(
                pe_hbm.at[pl.ds(i * chunk, chunk), :],
                bufs[i % nbuf], gsems[i % nbuf])

        def wcopy(i, b):
            if i == nch - 1:
                return pltpu.make_async_copy(
                    tbuf, out_hbm.at[b, pl.ds(i * chunk, tail), :],
                    wsems[i % nbuf])
            return pltpu.make_async_copy(
                bufs[i % nbuf],
                out_hbm.at[b, pl.ds(i * chunk, chunk), :],
                wsems[i % nbuf])

        gcopy(0).start()
        if nch > 1:
            gcopy(1).start()
        for i in range(nch):
            gcopy(i).wait()
            if i == nch - 1:
                tbuf[...] = bufs[i % nbuf][pl.ds(0, tail), :]
            for b in range(batch):
                wcopy(i, b).start()
            if i > 0:
                for b in range(batch):
                    wcopy(i - 1, b).wait()
            if i + 2 < nch:
                gcopy(i + 2).start()
        for b in range(batch):
            wcopy(nch - 1, b).wait()

    return pl.pallas_call(
        body,
        in_specs=[pl.BlockSpec(memory_space=pl.ANY)],
        out_specs=pl.BlockSpec(memory_space=pl.ANY),
        out_shape=jax.ShapeDtypeStruct((batch, length, d), jnp.float32),
        scratch_shapes=[
            [pltpu.VMEM((chunk, d), jnp.float32) for _ in range(nbuf)],
            pltpu.VMEM((tail, d), jnp.float32),
            [pltpu.SemaphoreType.DMA for _ in range(nbuf)],
            [pltpu.SemaphoreType.DMA for _ in range(nbuf)],
        ],
    )(pe2d)


def kernel(x, pe):
    b, s, _ = x.shape
    length = 2 * s - 1
    return _tc_broadcast_rows(pe[0], b, length)
